# baseline (speedup 1.0000x reference)
import functools

import jax
import jax.numpy as jnp
from jax import lax
from jax.experimental import pallas as pl
from jax.experimental.pallas import tpu as pltpu
from jax.experimental.pallas import tpu_sc as plsc

B = 4096
M = 50
P = 16
DIM = 64
NREL = 32

NC = 2
NS = 16
NW = NC * NS
RPW = B // NW
CB = 8
NCH = RPW // CB
EPC = CB * P
UPC = CB * M

_F32 = jnp.float32
_I32 = jnp.int32


def _mesh_kernel():
    mesh = plsc.VectorSubcoreMesh(core_axis_name="c", subcore_axis_name="s")

    @functools.partial(
        pl.kernel,
        mesh=mesh,
        out_type=jax.ShapeDtypeStruct((B,), _F32),
        compiler_params=pltpu.CompilerParams(
            needs_layout_passes=False, use_tc_tiling_on_sc=False),
        scratch_types=[
            pltpu.VMEM((NREL, DIM), _F32),
            pltpu.VMEM((8, 16), _F32),
            pltpu.VMEM((RPW,), _I32),
            pltpu.VMEM((RPW,), _I32),
            pltpu.VMEM((RPW, DIM), _F32),
            pltpu.VMEM((RPW, DIM), _F32),
            pltpu.VMEM((EPC + 16,), _I32),
            pltpu.VMEM((EPC + 16,), _I32),
            pltpu.VMEM((EPC + 16,), _I32),
            pltpu.VMEM((EPC + 16,), _I32),
            pltpu.VMEM((EPC + 16,), _I32),
            pltpu.VMEM((UPC,), _I32),
            pltpu.VMEM((UPC,), _I32),
            pltpu.VMEM((EPC, DIM), _F32),
            pltpu.VMEM((EPC, DIM), _F32),
            pltpu.VMEM((EPC, DIM), _F32),
            pltpu.VMEM((UPC, DIM), _F32),
            pltpu.VMEM((UPC, DIM), _F32),
            pltpu.VMEM((80,), _F32),
            pltpu.VMEM((RPW + 16,), _F32),
            pltpu.SemaphoreType.DMA,
        ],
    )
    def kern(ent1_i, ent0_i, head_i, rel1_i, rel0_i, utu_i, uti_i,
             users_i, items_i, uemb, eemb, remb, attw, out,
             rel_v, attw_v, uidx_v, iidx_v, urow_v, irow_v,
             ei1_v, ei0_v, eih_v, ri1_v, ri0_v, uui_v, uii_v,
             e1r_v, e0r_v, ehr_v, u0r_v, i0r_v, att_v, sc_v, sem):
        wid = lax.axis_index("s") * NC + lax.axis_index("c")
        base = wid * RPW

        pltpu.sync_copy(remb, rel_v)
        pltpu.sync_copy(attw, attw_v)
        pltpu.sync_copy(users_i.at[pl.ds(base, RPW)], uidx_v)
        pltpu.sync_copy(items_i.at[pl.ds(base, RPW)], iidx_v)
        pltpu.async_copy(uemb.at[uidx_v], urow_v, sem).wait()
        pltpu.async_copy(eemb.at[iidx_v], irow_v, sem).wait()

        wu = [attw_v[k] for k in range(4)]
        wi = [attw_v[k + 4] for k in range(4)]
        lane = lax.iota(_I32, 16)
        zero4 = (jnp.zeros((16,), _F32),) * 4
        neg4 = (jnp.full((16,), -1e30, _F32),) * 4

        def chunk(g, _):
            eb = (base + g * CB) * P
            ub = (base + g * CB) * M
            pltpu.sync_copy(ent1_i.at[pl.ds(eb, EPC)],
                            ei1_v.at[pl.ds(0, EPC)])
            pltpu.sync_copy(ent0_i.at[pl.ds(eb, EPC)],
                            ei0_v.at[pl.ds(0, EPC)])
            pltpu.sync_copy(head_i.at[pl.ds(eb, EPC)],
                            eih_v.at[pl.ds(0, EPC)])
            pltpu.sync_copy(rel1_i.at[pl.ds(eb, EPC)],
                            ri1_v.at[pl.ds(0, EPC)])
            pltpu.sync_copy(rel0_i.at[pl.ds(eb, EPC)],
                            ri0_v.at[pl.ds(0, EPC)])
            pltpu.sync_copy(utu_i.at[pl.ds(ub, UPC)], uui_v)
            pltpu.sync_copy(uti_i.at[pl.ds(ub, UPC)], uii_v)
            h1 = pltpu.async_copy(eemb.at[ei1_v.at[pl.ds(0, EPC)]],
                                  e1r_v, sem)
            h2 = pltpu.async_copy(eemb.at[ei0_v.at[pl.ds(0, EPC)]],
                                  e0r_v, sem)
            h3 = pltpu.async_copy(eemb.at[eih_v.at[pl.ds(0, EPC)]],
                                  ehr_v, sem)
            h4 = pltpu.async_copy(uemb.at[uui_v], u0r_v, sem)
            h5 = pltpu.async_copy(eemb.at[uii_v], i0r_v, sem)
            h1.wait(); h2.wait(); h3.wait(); h4.wait(); h5.wait()

            scv = jnp.zeros((16,), _F32)
            for b in range(CB):
                def prow(p, acc):
                    r = b * P + p
                    r1 = ri1_v[pl.ds(r, 16)][0]
                    r0 = ri0_v[pl.ds(r, 16)][0]
                    out_acc = []
                    for k in range(4):
                        sl = pl.ds(k * 16, 16)
                        w = rel_v[r1, sl] * e1r_v[r, sl]
                        w2 = (w + e0r_v[r, sl]) * rel_v[r0, sl]
                        out_acc.append(acc[k] + w2 + ehr_v[r, sl])
                    return tuple(out_acc)

                path = lax.fori_loop(0, P, prow, zero4, unroll=4)

                def matt(m, carry):
                    r = b * M + m
                    acc = u0r_v[r, pl.ds(0, 16)] * wu[0]
                    for k in range(1, 4):
                        acc = acc + u0r_v[r, pl.ds(k * 16, 16)] * wu[k]
                    for k in range(4):
                        acc = acc + i0r_v[r, pl.ds(k * 16, 16)] * wi[k]
                    t = jnp.sum(acc)
                    t = jnp.where(t > 0, t, t * 0.01)
                    sub = jnp.bitwise_and(m, 15)
                    which = m >> 4
                    return tuple(
                        jnp.where((which == k) & (lane == sub), t, carry[k])
                        for k in range(4))

                lg = list(lax.fori_loop(0, M, matt, neg4, unroll=5))

                mx = jnp.max(jnp.maximum(jnp.maximum(lg[0], lg[1]),
                                         jnp.maximum(lg[2], lg[3])))
                e = [jnp.exp(v - mx) for v in lg]
                inv = 1.0 / jnp.full((16,), jnp.sum(e[0] + e[1] + e[2] + e[3]),
                                     _F32)
                for k in range(4):
                    att_v[pl.ds(k * 16, 16)] = e[k] * inv

                def magg(m, acc):
                    r = b * M + m
                    w = att_v[pl.ds(m, 16)][0]
                    return tuple(
                        acc[k] + i0r_v[r, pl.ds(k * 16, 16)] * w
                        for k in range(4))

                agg = lax.fori_loop(0, M, magg, zero4, unroll=5)

                ro = g * CB + b
                d = jnp.zeros((16,), _F32)
                for k in range(4):
                    sl = pl.ds(k * 16, 16)
                    d = d + urow_v[ro, sl] * irow_v[ro, sl]
                    d = d + path[k] * (1.0 / (3.0 * P)) * agg[k]
                tot = jnp.sum(d)
                scv = jnp.where(lane == b, tot, scv)

            sig = 1.0 / (1.0 + jnp.exp(-scv))
            sc_v[pl.ds(g * CB, 16)] = sig
            return 0

        lax.fori_loop(0, NCH, chunk, 0)
        pltpu.sync_copy(sc_v.at[pl.ds(0, RPW)], out.at[pl.ds(base, RPW)])

    return kern


_KERN = _mesh_kernel()


def kernel(users, items, ut_user_idx, ut_item_idx, it_head, it_rel0,
           it_rel1, it_ent0, it_ent1, user_emb, entity_emb, relation_emb,
           att_weight):
    i32 = lambda x: x.astype(_I32)
    return _KERN(
        i32(it_ent1).reshape(B * P),
        i32(it_ent0).reshape(B * P),
        i32(it_head).reshape(B * P),
        i32(it_rel1).reshape(B * P),
        i32(it_rel0).reshape(B * P),
        i32(ut_user_idx).reshape(B * M),
        i32(ut_item_idx).reshape(B * M),
        i32(users),
        i32(items),
        user_emb,
        entity_emb,
        relation_emb,
        att_weight.reshape(8, 16),
    )


# speedup vs baseline: 1.5120x; 1.5120x over previous
import functools

import jax
import jax.numpy as jnp
from jax import lax
from jax.experimental import pallas as pl
from jax.experimental.pallas import tpu as pltpu
from jax.experimental.pallas import tpu_sc as plsc

B = 4096
M = 50
P = 16
DIM = 64
NREL = 32

NC = 2
NS = 16
NW = NC * NS
RPW = B // NW
CB = 4
NCH = RPW // CB
NPAIR = NCH // 2
EPC = CB * P
UPC = CB * M

_F32 = jnp.float32
_I32 = jnp.int32


def _mesh_kernel():
    mesh = plsc.VectorSubcoreMesh(core_axis_name="c", subcore_axis_name="s")

    idx_slot = [
        pltpu.VMEM((EPC + 16,), _I32),
        pltpu.VMEM((EPC + 16,), _I32),
        pltpu.VMEM((EPC + 16,), _I32),
        pltpu.VMEM((EPC + 16,), _I32),
        pltpu.VMEM((EPC + 16,), _I32),
        pltpu.VMEM((UPC,), _I32),
        pltpu.VMEM((UPC,), _I32),
    ]
    row_slot = [
        pltpu.VMEM((EPC, DIM), _F32),
        pltpu.VMEM((EPC, DIM), _F32),
        pltpu.VMEM((EPC, DIM), _F32),
        pltpu.VMEM((UPC, DIM), _F32),
        pltpu.VMEM((UPC, DIM), _F32),
    ]

    @functools.partial(
        pl.kernel,
        mesh=mesh,
        out_type=jax.ShapeDtypeStruct((B,), _F32),
        compiler_params=pltpu.CompilerParams(
            needs_layout_passes=False, use_tc_tiling_on_sc=False),
        scratch_types=[
            pltpu.VMEM((NREL, DIM), _F32),
            pltpu.VMEM((8, 16), _F32),
            pltpu.VMEM((RPW,), _I32),
            pltpu.VMEM((RPW,), _I32),
            pltpu.VMEM((RPW, DIM), _F32),
            pltpu.VMEM((RPW, DIM), _F32),
            pltpu.VMEM((80,), _F32),
            pltpu.VMEM((RPW + 16,), _F32),
        ] + idx_slot + idx_slot + row_slot + row_slot + [
            pltpu.SemaphoreType.DMA,
            pltpu.SemaphoreType.DMA,
            pltpu.SemaphoreType.DMA,
            pltpu.SemaphoreType.DMA,
            pltpu.SemaphoreType.DMA,
        ],
    )
    def kern(ent1_i, ent0_i, head_i, rel1_i, rel0_i, utu_i, uti_i,
             users_i, items_i, uemb, eemb, remb, attw, out,
             rel_v, attw_v, uidx_v, iidx_v, urow_v, irow_v, att_v, sc_v,
             *slot_refs):
        idxA = slot_refs[0:7]
        idxB = slot_refs[7:14]
        rowA = slot_refs[14:19]
        rowB = slot_refs[19:24]
        sem0, isemA, isemB, gsemA, gsemB = slot_refs[24:29]

        wid = lax.axis_index("s") * NC + lax.axis_index("c")
        base = wid * RPW

        pltpu.sync_copy(remb, rel_v)
        pltpu.sync_copy(attw, attw_v)
        pltpu.sync_copy(users_i.at[pl.ds(base, RPW)], uidx_v)
        pltpu.sync_copy(items_i.at[pl.ds(base, RPW)], iidx_v)
        pltpu.async_copy(uemb.at[uidx_v], urow_v, sem0).wait()
        pltpu.async_copy(eemb.at[iidx_v], irow_v, sem0).wait()

        idx_srcs = (ent1_i, ent0_i, head_i, rel1_i, rel0_i, utu_i, uti_i)

        def idx_copies(g, slot):
            eb = (base + g * CB) * P
            ub = (base + g * CB) * M
            outl = []
            for j in range(5):
                outl.append((idx_srcs[j].at[pl.ds(eb, EPC)],
                             slot[j].at[pl.ds(0, EPC)]))
            outl.append((idx_srcs[5].at[pl.ds(ub, UPC)], slot[5]))
            outl.append((idx_srcs[6].at[pl.ds(ub, UPC)], slot[6]))
            return outl

        def issue_idx(g, slot, sem):
            for src, dst in idx_copies(g, slot):
                pltpu.async_copy(src, dst, sem)

        def wait_idx(g, slot, sem):
            for src, dst in idx_copies(g, slot):
                pltpu.make_async_copy(src, dst, sem).wait()

        def gath_copies(islot, rslot):
            return [
                (eemb.at[islot[0].at[pl.ds(0, EPC)]], rslot[0]),
                (eemb.at[islot[1].at[pl.ds(0, EPC)]], rslot[1]),
                (eemb.at[islot[2].at[pl.ds(0, EPC)]], rslot[2]),
                (uemb.at[islot[5]], rslot[3]),
                (eemb.at[islot[6]], rslot[4]),
            ]

        def issue_gath(islot, rslot, sem):
            for src, dst in gath_copies(islot, rslot):
                pltpu.async_copy(src, dst, sem)

        def wait_gath(islot, rslot, sem):
            for src, dst in gath_copies(islot, rslot):
                pltpu.make_async_copy(src, dst, sem).wait()

        wu = [attw_v[k] for k in range(4)]
        wi = [attw_v[k + 4] for k in range(4)]
        lane = lax.iota(_I32, 16)
        zero4 = (jnp.zeros((16,), _F32),) * 4
        neg4 = (jnp.full((16,), -1e30, _F32),) * 4

        def compute(g, islot, rslot):
            ri1_v, ri0_v = islot[3], islot[4]
            e1r_v, e0r_v, ehr_v, u0r_v, i0r_v = rslot
            scv = jnp.zeros((16,), _F32)
            for b in range(CB):
                def prow(p, acc):
                    r = b * P + p
                    r1 = ri1_v[pl.ds(r, 16)][0]
                    r0 = ri0_v[pl.ds(r, 16)][0]
                    out_acc = []
                    for k in range(4):
                        sl = pl.ds(k * 16, 16)
                        w = rel_v[r1, sl] * e1r_v[r, sl]
                        w2 = (w + e0r_v[r, sl]) * rel_v[r0, sl]
                        out_acc.append(acc[k] + w2 + ehr_v[r, sl])
                    return tuple(out_acc)

                path = lax.fori_loop(0, P, prow, zero4)

                def matt(m, carry):
                    r = b * M + m
                    acc = u0r_v[r, pl.ds(0, 16)] * wu[0]
                    for k in range(1, 4):
                        acc = acc + u0r_v[r, pl.ds(k * 16, 16)] * wu[k]
                    for k in range(4):
                        acc = acc + i0r_v[r, pl.ds(k * 16, 16)] * wi[k]
                    t = jnp.sum(acc)
                    t = jnp.where(t > 0, t, t * 0.01)
                    sub = jnp.bitwise_and(m, 15)
                    which = m >> 4
                    return tuple(
                        jnp.where((which == k) & (lane == sub), t, carry[k])
                        for k in range(4))

                lg = list(lax.fori_loop(0, M, matt, neg4))

                mx = jnp.max(jnp.maximum(jnp.maximum(lg[0], lg[1]),
                                         jnp.maximum(lg[2], lg[3])))
                e = [jnp.exp(v - mx) for v in lg]
                inv = 1.0 / jnp.full((16,),
                                     jnp.sum(e[0] + e[1] + e[2] + e[3]), _F32)
                for k in range(4):
                    att_v[pl.ds(k * 16, 16)] = e[k] * inv

                def magg(m, acc):
                    r = b * M + m
                    w = att_v[pl.ds(m, 16)][0]
                    return tuple(
                        acc[k] + i0r_v[r, pl.ds(k * 16, 16)] * w
                        for k in range(4))

                agg = lax.fori_loop(0, M, magg, zero4)

                ro = g * CB + b
                d = jnp.zeros((16,), _F32)
                for k in range(4):
                    sl = pl.ds(k * 16, 16)
                    d = d + urow_v[ro, sl] * irow_v[ro, sl]
                    d = d + path[k] * (1.0 / (3.0 * P)) * agg[k]
                tot = jnp.sum(d)
                scv = jnp.where(lane == b, tot, scv)

            sig = 1.0 / (1.0 + jnp.exp(-scv))
            sc_v[pl.ds(g * CB, 16)] = sig

        issue_idx(0, idxA, isemA)
        wait_idx(0, idxA, isemA)
        issue_gath(idxA, rowA, gsemA)
        issue_idx(1, idxB, isemB)

        def pair(i, _):
            g0 = 2 * i
            g1 = g0 + 1
            wait_idx(g1, idxB, isemB)
            issue_gath(idxB, rowB, gsemB)
            wait_gath(idxA, rowA, gsemA)

            @pl.when(i < NPAIR - 1)
            def _():
                issue_idx(g0 + 2, idxA, isemA)

            compute(g0, idxA, rowA)

            @pl.when(i < NPAIR - 1)
            def _():
                wait_idx(g0 + 2, idxA, isemA)
                issue_gath(idxA, rowA, gsemA)

            wait_gath(idxB, rowB, gsemB)

            @pl.when(i < NPAIR - 1)
            def _():
                issue_idx(g1 + 2, idxB, isemB)

            compute(g1, idxB, rowB)
            return 0

        lax.fori_loop(0, NPAIR, pair, 0)
        pltpu.sync_copy(sc_v.at[pl.ds(0, RPW)], out.at[pl.ds(base, RPW)])

    return kern


_KERN = _mesh_kernel()


def kernel(users, items, ut_user_idx, ut_item_idx, it_head, it_rel0,
           it_rel1, it_ent0, it_ent1, user_emb, entity_emb, relation_emb,
           att_weight):
    i32 = lambda x: x.astype(_I32)
    return _KERN(
        i32(it_ent1).reshape(B * P),
        i32(it_ent0).reshape(B * P),
        i32(it_head).reshape(B * P),
        i32(it_rel1).reshape(B * P),
        i32(it_rel0).reshape(B * P),
        i32(ut_user_idx).reshape(B * M),
        i32(ut_item_idx).reshape(B * M),
        i32(users),
        i32(items),
        user_emb,
        entity_emb,
        relation_emb,
        att_weight.reshape(8, 16),
    )
